# baseline (speedup 1.0000x reference)
import functools

import jax
import jax.numpy as jnp
from jax import lax
from jax.experimental import pallas as pl
from jax.experimental.pallas import tpu as pltpu
from jax.experimental.pallas import tpu_sc as plsc

N = 10000
E = 320000
D = 128
H = 128
C = 10
G = 32

NC = 2
NS = 16
NW = NC * NS
EPW = E // NW
CHUNK = 128
NCHUNK = 80
EPP = NCHUNK * CHUNK
NP = 10240
RPP = NP // NS
DUMP = NP - 1
GRP = 8
NGRP = NCHUNK // GRP
NPAIR = NCHUNK // 2


@functools.cache
def _mesh():
    return plsc.VectorSubcoreMesh(
        core_axis_name="c", subcore_axis_name="s", num_cores=NC, num_subcores=NS
    )


def _deg_body(dst3, degp, shared, dbuf, ones_v, zb, sem):
    c = lax.axis_index("c")
    s = lax.axis_index("s")
    wid = c * NS + s

    ones16 = jnp.ones((16,), jnp.float32)
    zeros16 = jnp.zeros((16,), jnp.float32)

    def fill(i, _):
        ones_v[i, :] = ones16
        return 0

    lax.fori_loop(0, CHUNK, fill, 0)

    def fill2(i, _):
        zb[i, :] = zeros16
        return 0

    lax.fori_loop(0, RPP, fill2, 0)

    pltpu.sync_copy(zb, shared.at[pl.ds(s * RPP, RPP)])
    pltpu.sync_copy(dst3.at[wid], dbuf)
    plsc.subcore_barrier()

    def step(j, _):
        pltpu.async_copy(ones_v, shared.at[dbuf.at[j]], sem, add=True).wait()
        return 0

    lax.fori_loop(0, NCHUNK, step, 0)
    plsc.subcore_barrier()
    pltpu.sync_copy(shared.at[pl.ds(s * RPP, RPP)], degp.at[c, pl.ds(s * RPP, RPP)])


@functools.cache
def _deg_kernel():
  return pl.kernel(
    _deg_body,
    out_type=jax.ShapeDtypeStruct((NC, NP, 16), jnp.float32),
    mesh=_mesh(),
    scratch_types=[
        pltpu.VMEM_SHARED((NP, 16), jnp.float32),
        pltpu.VMEM((NCHUNK, CHUNK), jnp.int32),
        pltpu.VMEM((CHUNK, 16), jnp.float32),
        pltpu.VMEM((RPP, 16), jnp.float32),
        pltpu.SemaphoreType.DMA,
    ],
  )


def _agg_body(h, e3g, aggp, shared, ib, rb0, rb1, gs0, gs1, ss0, ss1):
    c = lax.axis_index("c")
    s = lax.axis_index("s")
    wid = c * NS + s

    zeros16 = jnp.zeros((16,), jnp.float32)

    def zrow(i, _):
        for k in range(D // 16):
            rb0[i, pl.ds(k * 16, 16)] = zeros16
        return 0

    lax.fori_loop(0, CHUNK, zrow, 0)

    for r in range(RPP // CHUNK):
        pltpu.async_copy(rb0, shared.at[pl.ds(s * RPP + r * CHUNK, CHUNK)], ss0)
    for r in range(RPP // CHUNK):
        pltpu.make_async_copy(rb0, shared.at[pl.ds(r * CHUNK, CHUNK)], ss0).wait()
    plsc.subcore_barrier()

    def group(g, _):
        pltpu.sync_copy(e3g.at[wid, g], ib)
        rbs = (rb0, rb1)
        gss = (gs0, gs1)
        sss = (ss0, ss1)
        pltpu.async_copy(h.at[ib.at[0]], rb0, gs0)
        pltpu.async_copy(h.at[ib.at[2]], rb1, gs1)
        for k in range(GRP):
            b = k % 2
            pltpu.make_async_copy(h.at[ib.at[2 * k]], rbs[b], gss[b]).wait()
            pltpu.async_copy(rbs[b], shared.at[ib.at[2 * k + 1]], sss[b], add=True)
            pltpu.make_async_copy(rbs[b], shared.at[ib.at[2 * k + 1]], sss[b]).wait()
            if k + 2 < GRP:
                pltpu.async_copy(h.at[ib.at[2 * (k + 2)]], rbs[b], gss[b])
        return 0

    lax.fori_loop(0, NGRP, group, 0)
    plsc.subcore_barrier()

    pltpu.sync_copy(shared.at[pl.ds(s * RPP, RPP)], aggp.at[c, pl.ds(s * RPP, RPP)])


@functools.cache
def _agg_kernel():
  return pl.kernel(
    _agg_body,
    out_type=jax.ShapeDtypeStruct((NC, NP, H), jnp.float32),
    mesh=_mesh(),
    scratch_types=[
        pltpu.VMEM_SHARED((NP, H), jnp.float32),
        pltpu.VMEM((2 * GRP, CHUNK), jnp.int32),
        pltpu.VMEM((CHUNK, H), jnp.float32),
        pltpu.VMEM((CHUNK, H), jnp.float32),
        pltpu.SemaphoreType.DMA,
        pltpu.SemaphoreType.DMA,
        pltpu.SemaphoreType.DMA,
        pltpu.SemaphoreType.DMA,
    ],
  )


BLK = 400
NBLK = N // BLK


def _dinv_from_degp(degp_blk):
    deg = degp_blk[0, :, 0] + degp_blk[1, :, 0] + 1.0
    return lax.rsqrt(deg)


def _mm1_body(degp_ref, x_ref, w_ref, o_ref):
    dinv = _dinv_from_degp(degp_ref[...])
    xs = x_ref[...] * dinv[:, None]
    o_ref[...] = jnp.dot(xs, w_ref[...], preferred_element_type=jnp.float32,
                         precision=lax.Precision.HIGHEST)


def _mm2_body(degp_ref, aggp_ref, hp_ref, b_ref, w_ref, o_ref):
    dinv = _dinv_from_degp(degp_ref[...])
    tot = aggp_ref[0] + aggp_ref[1] + hp_ref[...]
    z = jnp.maximum(tot * dinv[:, None] + b_ref[...], 0.0)
    o_ref[...] = jnp.dot(z * dinv[:, None], w_ref[...],
                         preferred_element_type=jnp.float32,
                         precision=lax.Precision.HIGHEST)


def _final_body(degp_ref, aggp_ref, hp_ref, b_ref, batch_ref, wlin_ref,
                blin_ref, o_ref, acc, cnt):
    i = pl.program_id(0)

    @pl.when(i == 0)
    def _():
        acc[...] = jnp.zeros_like(acc)
        cnt[...] = jnp.zeros_like(cnt)

    dinv = _dinv_from_degp(degp_ref[...])
    tot = aggp_ref[0] + aggp_ref[1] + hp_ref[...]
    z = jnp.maximum(tot * dinv[:, None] + b_ref[...], 0.0)

    gids = lax.broadcasted_iota(jnp.int32, (G, BLK), 0)
    m = (batch_ref[0] == gids).astype(jnp.float32)
    acc[...] += jnp.dot(m, z, preferred_element_type=jnp.float32,
                        precision=lax.Precision.HIGHEST)
    cnt[...] += jnp.broadcast_to(jnp.sum(m, axis=1, keepdims=True), (G, H))

    @pl.when(i == NBLK - 1)
    def _():
        pooled = acc[...] / jnp.maximum(cnt[...], 1.0)
        o_ref[...] = jnp.dot(pooled, wlin_ref[...],
                             preferred_element_type=jnp.float32,
                             precision=lax.Precision.HIGHEST) + blin_ref[...]


def _degp_spec():
    return pl.BlockSpec((NC, BLK, 16), lambda i: (0, i, 0))


_mm1 = pl.pallas_call(
    _mm1_body,
    grid=(NBLK,),
    in_specs=[
        _degp_spec(),
        pl.BlockSpec((BLK, D), lambda i: (i, 0)),
        pl.BlockSpec((D, H), lambda i: (0, 0)),
    ],
    out_specs=pl.BlockSpec((BLK, H), lambda i: (i, 0)),
    out_shape=jax.ShapeDtypeStruct((N, H), jnp.float32),
)

_mm2 = pl.pallas_call(
    _mm2_body,
    grid=(NBLK,),
    in_specs=[
        _degp_spec(),
        pl.BlockSpec((NC, BLK, H), lambda i: (0, i, 0)),
        pl.BlockSpec((BLK, H), lambda i: (i, 0)),
        pl.BlockSpec((1, H), lambda i: (0, 0)),
        pl.BlockSpec((H, H), lambda i: (0, 0)),
    ],
    out_specs=pl.BlockSpec((BLK, H), lambda i: (i, 0)),
    out_shape=jax.ShapeDtypeStruct((N, H), jnp.float32),
)

_final = pl.pallas_call(
    _final_body,
    grid=(NBLK,),
    in_specs=[
        _degp_spec(),
        pl.BlockSpec((NC, BLK, H), lambda i: (0, i, 0)),
        pl.BlockSpec((BLK, H), lambda i: (i, 0)),
        pl.BlockSpec((1, H), lambda i: (0, 0)),
        pl.BlockSpec((1, 1, BLK), lambda i: (i, 0, 0)),
        pl.BlockSpec((H, C), lambda i: (0, 0)),
        pl.BlockSpec((1, C), lambda i: (0, 0)),
    ],
    out_specs=pl.BlockSpec((G, C), lambda i: (0, 0)),
    out_shape=jax.ShapeDtypeStruct((G, C), jnp.float32),
    scratch_shapes=[
        pltpu.VMEM((G, H), jnp.float32),
        pltpu.VMEM((G, H), jnp.float32),
    ],
)


@jax.jit
def kernel(x, edge_index, batch, W1, b1, W2, b2, Wlin, blin):
    pad = EPP - EPW
    srcp = jnp.pad(edge_index[0].reshape(NW, EPW), ((0, 0), (0, pad)))
    dstp = jnp.pad(edge_index[1].reshape(NW, EPW), ((0, 0), (0, pad)),
                   constant_values=DUMP)
    src3 = srcp.reshape(NW, NCHUNK, CHUNK)
    dst3 = dstp.reshape(NW, NCHUNK, CHUNK)
    e4 = jnp.stack([src3, dst3], axis=2)
    e3g = e4.reshape(NW, NGRP, 2 * GRP, CHUNK)

    degp = _deg_kernel()(dst3)
    h1p = _mm1(degp, x, W1)
    agg1 = _agg_kernel()(h1p, e3g)
    h2p = _mm2(degp, agg1, h1p, b1.reshape(1, H), W2)
    agg2 = _agg_kernel()(h2p, e3g)
    return _final(degp, agg2, h2p, b2.reshape(1, H),
                  batch.reshape(NBLK, 1, BLK), Wlin, blin.reshape(1, C))


# speedup vs baseline: 1.0182x; 1.0182x over previous
import functools

import jax
import jax.numpy as jnp
from jax import lax
from jax.experimental import pallas as pl
from jax.experimental.pallas import tpu as pltpu
from jax.experimental.pallas import tpu_sc as plsc

N = 10000
E = 320000
D = 128
H = 128
C = 10
G = 32

NC = 2
NS = 16
NW = NC * NS
EPW = E // NW
CHUNK = 128
NCHUNK = 80
EPP = NCHUNK * CHUNK
NP = 10240
RPP = NP // NS
DUMP = NP - 1
GRP = 16
NGRP = NCHUNK // GRP
NPAIR = NCHUNK // 2


@functools.cache
def _mesh():
    return plsc.VectorSubcoreMesh(
        core_axis_name="c", subcore_axis_name="s", num_cores=NC, num_subcores=NS
    )


def _deg_body(dst3, degp, shared, dbuf, ones_v, zb, sem):
    c = lax.axis_index("c")
    s = lax.axis_index("s")
    wid = c * NS + s

    ones16 = jnp.ones((16,), jnp.float32)
    zeros16 = jnp.zeros((16,), jnp.float32)

    def fill(i, _):
        ones_v[i, :] = ones16
        return 0

    lax.fori_loop(0, CHUNK, fill, 0)

    def fill2(i, _):
        zb[i, :] = zeros16
        return 0

    lax.fori_loop(0, RPP, fill2, 0)

    pltpu.sync_copy(zb, shared.at[pl.ds(s * RPP, RPP)])
    pltpu.sync_copy(dst3.at[wid], dbuf)
    plsc.subcore_barrier()

    def step(j, _):
        pltpu.async_copy(ones_v, shared.at[dbuf.at[j]], sem, add=True).wait()
        return 0

    lax.fori_loop(0, NCHUNK, step, 0)
    plsc.subcore_barrier()
    pltpu.sync_copy(shared.at[pl.ds(s * RPP, RPP)], degp.at[c, pl.ds(s * RPP, RPP)])


@functools.cache
def _deg_kernel():
  return pl.kernel(
    _deg_body,
    out_type=jax.ShapeDtypeStruct((NC, NP, 16), jnp.float32),
    mesh=_mesh(),
    scratch_types=[
        pltpu.VMEM_SHARED((NP, 16), jnp.float32),
        pltpu.VMEM((NCHUNK, CHUNK), jnp.int32),
        pltpu.VMEM((CHUNK, 16), jnp.float32),
        pltpu.VMEM((RPP, 16), jnp.float32),
        pltpu.SemaphoreType.DMA,
    ],
  )


def _agg_body(h, e3g, aggp, shared, ib, rb0, rb1, gs0, gs1, ss0, ss1):
    c = lax.axis_index("c")
    s = lax.axis_index("s")
    wid = c * NS + s

    zeros16 = jnp.zeros((16,), jnp.float32)

    def zrow(i, _):
        for k in range(D // 16):
            rb0[i, pl.ds(k * 16, 16)] = zeros16
        return 0

    lax.fori_loop(0, CHUNK, zrow, 0)

    for r in range(RPP // CHUNK):
        pltpu.sync_copy(rb0, shared.at[pl.ds(s * RPP + r * CHUNK, CHUNK)])
    plsc.subcore_barrier()

    def group(g, _):
        pltpu.sync_copy(e3g.at[wid, g], ib)
        rbs = (rb0, rb1)
        gss = (gs0, gs1)
        sss = (ss0, ss1)
        pltpu.async_copy(h.at[ib.at[0]], rb0, gs0)
        pltpu.async_copy(h.at[ib.at[2]], rb1, gs1)
        for k in range(GRP):
            b = k % 2
            pltpu.make_async_copy(h.at[ib.at[2 * k]], rbs[b], gss[b]).wait()
            pltpu.async_copy(rbs[b], shared.at[ib.at[2 * k + 1]], sss[b], add=True)
            pltpu.make_async_copy(rbs[b], shared.at[ib.at[2 * k + 1]], sss[b]).wait()
            if k + 2 < GRP:
                pltpu.async_copy(h.at[ib.at[2 * (k + 2)]], rbs[b], gss[b])
        return 0

    lax.fori_loop(0, NGRP, group, 0)
    plsc.subcore_barrier()

    pltpu.sync_copy(shared.at[pl.ds(s * RPP, RPP)], aggp.at[c, pl.ds(s * RPP, RPP)])


@functools.cache
def _agg_kernel():
  return pl.kernel(
    _agg_body,
    out_type=jax.ShapeDtypeStruct((NC, NP, H), jnp.float32),
    mesh=_mesh(),
    scratch_types=[
        pltpu.VMEM_SHARED((NP, H), jnp.float32),
        pltpu.VMEM((2 * GRP, CHUNK), jnp.int32),
        pltpu.VMEM((CHUNK, H), jnp.float32),
        pltpu.VMEM((CHUNK, H), jnp.float32),
        pltpu.SemaphoreType.DMA,
        pltpu.SemaphoreType.DMA,
        pltpu.SemaphoreType.DMA,
        pltpu.SemaphoreType.DMA,
    ],
  )


BLK = 400
NBLK = N // BLK


def _dinv_from_degp(degp_blk):
    deg = degp_blk[0, :, 0] + degp_blk[1, :, 0] + 1.0
    return lax.rsqrt(deg)


def _mm1_body(degp_ref, x_ref, w_ref, o_ref):
    dinv = _dinv_from_degp(degp_ref[...])
    xs = x_ref[...] * dinv[:, None]
    o_ref[...] = jnp.dot(xs, w_ref[...], preferred_element_type=jnp.float32,
                         precision=lax.Precision.HIGHEST)


def _mm2_body(degp_ref, aggp_ref, hp_ref, b_ref, w_ref, o_ref):
    dinv = _dinv_from_degp(degp_ref[...])
    tot = aggp_ref[0] + aggp_ref[1] + hp_ref[...]
    z = jnp.maximum(tot * dinv[:, None] + b_ref[...], 0.0)
    o_ref[...] = jnp.dot(z * dinv[:, None], w_ref[...],
                         preferred_element_type=jnp.float32,
                         precision=lax.Precision.HIGHEST)


def _final_body(degp_ref, aggp_ref, hp_ref, b_ref, batch_ref, wlin_ref,
                blin_ref, o_ref, acc, cnt):
    i = pl.program_id(0)

    @pl.when(i == 0)
    def _():
        acc[...] = jnp.zeros_like(acc)
        cnt[...] = jnp.zeros_like(cnt)

    dinv = _dinv_from_degp(degp_ref[...])
    tot = aggp_ref[0] + aggp_ref[1] + hp_ref[...]
    z = jnp.maximum(tot * dinv[:, None] + b_ref[...], 0.0)

    gids = lax.broadcasted_iota(jnp.int32, (G, BLK), 0)
    m = (batch_ref[0] == gids).astype(jnp.float32)
    acc[...] += jnp.dot(m, z, preferred_element_type=jnp.float32,
                        precision=lax.Precision.HIGHEST)
    cnt[...] += jnp.broadcast_to(jnp.sum(m, axis=1, keepdims=True), (G, H))

    @pl.when(i == NBLK - 1)
    def _():
        pooled = acc[...] / jnp.maximum(cnt[...], 1.0)
        o_ref[...] = jnp.dot(pooled, wlin_ref[...],
                             preferred_element_type=jnp.float32,
                             precision=lax.Precision.HIGHEST) + blin_ref[...]


def _degp_spec():
    return pl.BlockSpec((NC, BLK, 16), lambda i: (0, i, 0))


_mm1 = pl.pallas_call(
    _mm1_body,
    grid=(NBLK,),
    in_specs=[
        _degp_spec(),
        pl.BlockSpec((BLK, D), lambda i: (i, 0)),
        pl.BlockSpec((D, H), lambda i: (0, 0)),
    ],
    out_specs=pl.BlockSpec((BLK, H), lambda i: (i, 0)),
    out_shape=jax.ShapeDtypeStruct((N, H), jnp.float32),
)

_mm2 = pl.pallas_call(
    _mm2_body,
    grid=(NBLK,),
    in_specs=[
        _degp_spec(),
        pl.BlockSpec((NC, BLK, H), lambda i: (0, i, 0)),
        pl.BlockSpec((BLK, H), lambda i: (i, 0)),
        pl.BlockSpec((1, H), lambda i: (0, 0)),
        pl.BlockSpec((H, H), lambda i: (0, 0)),
    ],
    out_specs=pl.BlockSpec((BLK, H), lambda i: (i, 0)),
    out_shape=jax.ShapeDtypeStruct((N, H), jnp.float32),
)

_final = pl.pallas_call(
    _final_body,
    grid=(NBLK,),
    in_specs=[
        _degp_spec(),
        pl.BlockSpec((NC, BLK, H), lambda i: (0, i, 0)),
        pl.BlockSpec((BLK, H), lambda i: (i, 0)),
        pl.BlockSpec((1, H), lambda i: (0, 0)),
        pl.BlockSpec((1, 1, BLK), lambda i: (i, 0, 0)),
        pl.BlockSpec((H, C), lambda i: (0, 0)),
        pl.BlockSpec((1, C), lambda i: (0, 0)),
    ],
    out_specs=pl.BlockSpec((G, C), lambda i: (0, 0)),
    out_shape=jax.ShapeDtypeStruct((G, C), jnp.float32),
    scratch_shapes=[
        pltpu.VMEM((G, H), jnp.float32),
        pltpu.VMEM((G, H), jnp.float32),
    ],
)


@jax.jit
def kernel(x, edge_index, batch, W1, b1, W2, b2, Wlin, blin):
    pad = EPP - EPW
    srcp = jnp.pad(edge_index[0].reshape(NW, EPW), ((0, 0), (0, pad)))
    dstp = jnp.pad(edge_index[1].reshape(NW, EPW), ((0, 0), (0, pad)),
                   constant_values=DUMP)
    src3 = srcp.reshape(NW, NCHUNK, CHUNK)
    dst3 = dstp.reshape(NW, NCHUNK, CHUNK)
    e4 = jnp.stack([src3, dst3], axis=2)
    e3g = e4.reshape(NW, NGRP, 2 * GRP, CHUNK)

    degp = _deg_kernel()(dst3)
    h1p = _mm1(degp, x, W1)
    agg1 = _agg_kernel()(h1p, e3g)
    h2p = _mm2(degp, agg1, h1p, b1.reshape(1, H), W2)
    agg2 = _agg_kernel()(h2p, e3g)
    return _final(degp, agg2, h2p, b2.reshape(1, H),
                  batch.reshape(NBLK, 1, BLK), Wlin, blin.reshape(1, C))
